# Initial kernel scaffold; baseline (speedup 1.0000x reference)
import jax
import jax.numpy as jnp
from jax.experimental import pallas as pl


def kernel(x, gate_w, w1, w2, w3):
    raise NotImplementedError("write your pallas kernel here")


# speedup vs baseline: 1.1080x; 1.1080x over previous
import functools

import jax
import jax.numpy as jnp
from jax import lax
from jax.experimental import pallas as pl
from jax.experimental.pallas import tpu as pltpu
from jax.experimental.pallas import tpu_sc as plsc

BB = 128


def _make_router(T, D, E, interpret=False):
    def body(x_ref, gw_ref, selw_ref, sel_ref):
        xb = x_ref[...]
        gw = gw_ref[...]
        logits = lax.dot_general(xb, gw, (((1,), (1,)), ((), ())),
                                 preferred_element_type=jnp.float32)
        m = jnp.max(logits, axis=1, keepdims=True)
        ex = jnp.exp(logits - m)
        p = ex / jnp.sum(ex, axis=1, keepdims=True)
        io = lax.broadcasted_iota(jnp.int32, (T, E), 1)
        m1 = jnp.max(p, axis=1, keepdims=True)
        e1 = jnp.min(jnp.where(p == m1, io, E), axis=1, keepdims=True)
        pm = jnp.where(io == e1, jnp.float32(-1.0), p)
        m2 = jnp.max(pm, axis=1, keepdims=True)
        e2 = jnp.min(jnp.where(pm == m2, io, E), axis=1, keepdims=True)
        s = m1 + m2
        selw_ref[...] = jnp.concatenate([m1 / s, m2 / s], axis=1)
        sel_ref[...] = jnp.concatenate([e1, e2], axis=1).astype(jnp.int32)

    return pl.pallas_call(
        body,
        out_shape=(jax.ShapeDtypeStruct((T, 2), jnp.float32),
                   jax.ShapeDtypeStruct((T, 2), jnp.int32)),
        interpret=interpret,
    )


def _make_ffn(NB, D, F, interpret=False):
    def body(be_ref, nbu_ref, xs_ref, w1_ref, w2_ref, w3_ref, rw_ref, out_ref):
        i = pl.program_id(0)

        @pl.when(i < nbu_ref[0])
        def _():
            xb = xs_ref[...]
            h1 = lax.dot_general(xb, w1_ref[0], (((1,), (1,)), ((), ())),
                                 preferred_element_type=jnp.float32)
            h3 = lax.dot_general(xb, w3_ref[0], (((1,), (1,)), ((), ())),
                                 preferred_element_type=jnp.float32)
            h = jnp.maximum(h1, 0.0) * h3
            ob = lax.dot_general(h, w2_ref[0], (((1,), (1,)), ((), ())),
                                 preferred_element_type=jnp.float32)
            out_ref[...] = ob * rw_ref[0]

    grid_spec = pltpu.PrefetchScalarGridSpec(
        num_scalar_prefetch=2,
        grid=(NB,),
        in_specs=[
            pl.BlockSpec((BB, D), lambda i, be, nbu: (i, 0)),
            pl.BlockSpec((1, F, D), lambda i, be, nbu: (be[i], 0, 0)),
            pl.BlockSpec((1, D, F), lambda i, be, nbu: (be[i], 0, 0)),
            pl.BlockSpec((1, F, D), lambda i, be, nbu: (be[i], 0, 0)),
            pl.BlockSpec((1, BB, 1), lambda i, be, nbu: (i, 0, 0)),
        ],
        out_specs=pl.BlockSpec((BB, D), lambda i, be, nbu: (i, 0)),
    )
    return pl.pallas_call(
        body,
        grid_spec=grid_spec,
        out_shape=jax.ShapeDtypeStruct((NB * BB, D), jnp.float32),
        interpret=interpret,
    )


def _make_sc_gather(D, PT):
    info = plsc.get_sparse_core_info()
    NC, NS = info.num_cores, info.num_subcores
    NW = NC * NS
    R = PT // NW
    CH = 32
    NCH = R // CH
    mesh = plsc.VectorSubcoreMesh(core_axis_name="c", subcore_axis_name="s")

    @functools.partial(
        pl.kernel, mesh=mesh,
        out_type=jax.ShapeDtypeStruct((PT, D), jnp.float32),
        scratch_types=[
            pltpu.VMEM((R,), jnp.int32),
            pltpu.VMEM((CH, D), jnp.float32),
            pltpu.VMEM((CH, D), jnp.float32),
            pltpu.SemaphoreType.DMA,
            pltpu.SemaphoreType.DMA,
        ],
    )
    def k(x_hbm, idx_hbm, xs_hbm, idx_v, buf0, buf1, sem0, sem1):
        wid = lax.axis_index("s") * NC + lax.axis_index("c")
        base = wid * R
        pltpu.sync_copy(idx_hbm.at[pl.ds(base, R)], idx_v)
        bufs = (buf0, buf1)
        sems = (sem0, sem1)
        copies = [None] * NCH
        copies[0] = pltpu.async_copy(
            x_hbm.at[idx_v.at[pl.ds(0, CH)]], bufs[0], sems[0])
        for c in range(NCH):
            if c + 1 < NCH:
                copies[c + 1] = pltpu.async_copy(
                    x_hbm.at[idx_v.at[pl.ds((c + 1) * CH, CH)]],
                    bufs[(c + 1) % 2], sems[(c + 1) % 2])
            copies[c].wait()
            pltpu.sync_copy(bufs[c % 2], xs_hbm.at[pl.ds(base + c * CH, CH)])

    return k


def _make_sc_combine(T, D, PT):
    info = plsc.get_sparse_core_info()
    NC, NS = info.num_cores, info.num_subcores
    NW = NC * NS
    TW = T // NW
    CT = 16
    NCH = TW // CT
    RCH = 2 * CT
    mesh = plsc.VectorSubcoreMesh(core_axis_name="c", subcore_axis_name="s")

    @functools.partial(
        pl.kernel, mesh=mesh,
        out_type=jax.ShapeDtypeStruct((T, D), jnp.float32),
        scratch_types=[
            pltpu.VMEM((2 * TW,), jnp.int32),
            pltpu.VMEM((RCH, D), jnp.float32),
            pltpu.VMEM((RCH, D), jnp.float32),
            pltpu.VMEM((CT, D), jnp.float32),
            pltpu.SemaphoreType.DMA,
            pltpu.SemaphoreType.DMA,
        ],
    )
    def k(ys_hbm, ridx_hbm, out_hbm, ridx_v, g0, g1, o_v, sem0, sem1):
        wid = lax.axis_index("s") * NC + lax.axis_index("c")
        base = wid * TW
        pltpu.sync_copy(ridx_hbm.at[pl.ds(wid * 2 * TW, 2 * TW)], ridx_v)
        bufs = (g0, g1)
        sems = (sem0, sem1)
        copies = [None] * NCH
        copies[0] = pltpu.async_copy(
            ys_hbm.at[ridx_v.at[pl.ds(0, RCH)]], bufs[0], sems[0])
        for c in range(NCH):
            if c + 1 < NCH:
                copies[c + 1] = pltpu.async_copy(
                    ys_hbm.at[ridx_v.at[pl.ds((c + 1) * RCH, RCH)]],
                    bufs[(c + 1) % 2], sems[(c + 1) % 2])
            copies[c].wait()
            g = bufs[c % 2]

            def row_body(j, _, g=g):
                for c2 in range(D // 16):
                    sl = pl.ds(c2 * 16, 16)
                    o_v[j, sl] = g[2 * j, sl] + g[2 * j + 1, sl]
                return 0

            lax.fori_loop(0, CT, row_body, 0)
            pltpu.sync_copy(o_v, out_hbm.at[pl.ds(base + c * CT, CT)])

    return k


def kernel(x, gate_w, w1, w2, w3):
    Bb, S, D = x.shape
    T = Bb * S
    E, F, _ = w1.shape
    K = 2
    NP = T * K
    NB = NP // BB + E
    PT = NB * BB

    xf = x.reshape(T, D)

    selw, sel = _make_router(T, D, E)(xf, gate_w)

    ep = sel.reshape(NP)
    pw = selw.reshape(NP)
    oh = (ep[:, None] == jnp.arange(E, dtype=jnp.int32)[None, :]).astype(jnp.int32)
    cum = jnp.cumsum(oh, axis=0)
    counts = cum[-1]
    rank = jnp.take_along_axis(cum, ep[:, None], axis=1)[:, 0] - 1
    blocks_e = (counts + BB - 1) // BB
    bcum = jnp.cumsum(blocks_e)
    poff = (bcum - blocks_e) * BB
    pos = poff[ep] + rank
    row_token = jnp.zeros((PT,), jnp.int32).at[pos].set(
        (jnp.arange(NP, dtype=jnp.int32) // K))
    row_w = jnp.zeros((PT,), jnp.float32).at[pos].set(pw)
    nbu = bcum[-1].astype(jnp.int32)[None]
    be = jnp.minimum(
        jnp.searchsorted(bcum, jnp.arange(NB, dtype=jnp.int32), side="right"),
        E - 1).astype(jnp.int32)

    xs = _make_sc_gather(D, PT)(xf, row_token)

    rw3 = row_w.reshape(NB, BB, 1)
    ys = _make_ffn(NB, D, F)(be, nbu, xs, w1, w2, w3, rw3)

    out = _make_sc_combine(T, D, PT)(ys, pos.astype(jnp.int32))
    return out.reshape(Bb, S, D)
